# Initial kernel scaffold; baseline (speedup 1.0000x reference)
import jax
import jax.numpy as jnp
from jax.experimental import pallas as pl


def kernel(x, mesh, W, b):
    raise NotImplementedError("write your pallas kernel here")


# speedup vs baseline: 10.7628x; 10.7628x over previous
import functools

import jax
import jax.numpy as jnp
from jax import lax
from jax.experimental import pallas as pl
from jax.experimental.pallas import tpu as pltpu
from jax.experimental.pallas import tpu_sc as plsc

_NC = 2
_NS = 16
_NW = _NC * _NS
_L = 16


def _make_mean_kernel(B, V, Vp, C, K, CH):
    total = B * Vp
    per_w = total // _NW
    n_ch = per_w // CH
    assert per_w * _NW == total and n_ch * CH == per_w
    assert Vp % per_w == 0
    assert CH * K <= 128
    assert CH % 8 == 0
    assert C % _L == 0 and (K * CH) % _L == 0

    mesh_axes = plsc.VectorSubcoreMesh(core_axis_name="c", subcore_axis_name="s")

    @functools.partial(
        pl.kernel,
        mesh=mesh_axes,
        out_type=jax.ShapeDtypeStruct((total, C), jnp.float32),
        scratch_types=[
            pltpu.VMEM((CH * K,), jnp.int32),
            pltpu.VMEM((CH * K, C), jnp.float32),
            pltpu.VMEM((CH, C), jnp.float32),
            pltpu.SemaphoreType.DMA,
        ],
    )
    def mean_kernel(x_hbm, idx_hbm, out_hbm, idx_v, rows_v, out_v, sem):
        wid = lax.axis_index("s") * _NC + lax.axis_index("c")
        vbase = wid * per_w
        boff = (vbase // Vp) * V
        offv = jnp.full((_L,), boff, jnp.int32)

        def body(ch, carry):
            v0 = vbase + ch * CH
            pltpu.sync_copy(idx_hbm.at[pl.ds(v0 * K, CH * K)], idx_v)
            for j in range(CH * K // _L):
                sl = pl.ds(j * _L, _L)
                idx_v[sl] = idx_v[sl] + offv
            pltpu.async_copy(x_hbm.at[idx_v], rows_v, sem).wait()
            for v in range(CH):
                for c in range(C // _L):
                    sl = pl.ds(c * _L, _L)
                    s = rows_v[v * K, sl]
                    for k in range(1, K):
                        s = s + rows_v[v * K + k, sl]
                    out_v[v, sl] = s * (1.0 / K)
            pltpu.sync_copy(out_v, out_hbm.at[pl.ds(v0, CH)])
            return carry

        lax.fori_loop(0, n_ch, body, 0)

    return mean_kernel


def _conv_body(x_ref, m_ref, w0t_ref, w1t_ref, b_ref, o_ref):
    acc = lax.dot_general(
        x_ref[0], w0t_ref[...], (((1,), (0,)), ((), ())),
        preferred_element_type=jnp.float32)
    acc = acc + lax.dot_general(
        m_ref[0], w1t_ref[...], (((1,), (0,)), ((), ())),
        preferred_element_type=jnp.float32)
    o_ref[0] = acc + b_ref[...]


def _tc_conv(x_t, mean_pad, w0t, w1t, b2d, BV=2000):
    B, V, C = x_t.shape
    Cout = w0t.shape[1]
    grid = (B, V // BV)
    return pl.pallas_call(
        _conv_body,
        grid=grid,
        in_specs=[
            pl.BlockSpec((1, BV, C), lambda i, j: (i, j, 0)),
            pl.BlockSpec((1, BV, C), lambda i, j: (i, j, 0)),
            pl.BlockSpec((C, Cout), lambda i, j: (0, 0)),
            pl.BlockSpec((C, Cout), lambda i, j: (0, 0)),
            pl.BlockSpec((1, Cout), lambda i, j: (0, 0)),
        ],
        out_specs=pl.BlockSpec((1, BV, Cout), lambda i, j: (i, j, 0)),
        out_shape=jax.ShapeDtypeStruct((B, V, Cout), jnp.float32),
    )(x_t, mean_pad, w0t, w1t, b2d)


def kernel(x, mesh, W, b):
    B, C, V = x.shape
    K = mesh.shape[-1]
    Vp = 10240 if V == 10000 else ((V + _NW * 8 - 1) // (_NW * 8)) * (_NW * 8)
    x_t = jnp.transpose(x, (0, 2, 1))
    mesh_pad = jnp.pad(mesh, ((0, 0), (0, Vp - V), (0, 0)))
    mean_pad = _make_mean_kernel(B, V, Vp, C, K, CH=8)(
        x_t.reshape(B * V, C), mesh_pad.reshape(B * Vp * K))
    w0t = jnp.transpose(W[:, :, 0, 0])
    w1t = jnp.transpose(W[:, :, 0, 1])
    out_t = _tc_conv(x_t, mean_pad.reshape(B, Vp, C), w0t, w1t,
                     b.reshape(1, -1))
    out = jnp.transpose(out_t, (0, 2, 1))
    return out[..., None]


# speedup vs baseline: 25.1716x; 2.3388x over previous
import functools

import numpy as np

import jax
import jax.numpy as jnp
from jax import lax
from jax.experimental import pallas as pl
from jax.experimental.pallas import tpu as pltpu
from jax.experimental.pallas import tpu_sc as plsc

_NC = 2
_NS = 16
_NW = _NC * _NS
_L = 16
_CH = 8
_NBUF = 2


def _make_mean_kernel(B, V, Vp, C, K):
    total = B * Vp
    per_w = total // _NW
    n_ch = per_w // _CH
    chk = _CH * K
    assert per_w * _NW == total and n_ch * _CH == per_w
    assert Vp % per_w == 0
    assert chk <= 128
    assert n_ch % _NBUF == 0 and C % 32 == 0
    assert (_NBUF * _CH) % 16 == 0

    mesh_axes = plsc.VectorSubcoreMesh(core_axis_name="c", subcore_axis_name="s")
    C2 = C // 2

    @functools.partial(
        pl.kernel,
        mesh=mesh_axes,
        compiler_params=pltpu.CompilerParams(use_tc_tiling_on_sc=False),
        out_type=jax.ShapeDtypeStruct((total, C), jnp.float32),
        scratch_types=[
            pltpu.VMEM((per_w * K,), jnp.int32),
            pltpu.VMEM((_NBUF, chk, C2), jnp.int32),
            pltpu.VMEM((_NBUF * _CH, C), jnp.float32),
            pltpu.SemaphoreType.DMA,
            pltpu.SemaphoreType.DMA,
        ],
    )
    def mean_kernel(x_hbm, idx_hbm, out_hbm, idx_v, rows_v, out_v, s0, s1):
        wid = lax.axis_index("s") * _NC + lax.axis_index("c")
        vbase = wid * per_w
        batch = vbase // Vp
        xb = x_hbm.at[pl.ds(batch * Vp, Vp)]
        sems = (s0, s1)

        pltpu.sync_copy(idx_hbm.at[pl.ds(vbase * K, per_w * K)], idx_v)

        def gather(ch, buf, sem):
            pltpu.async_copy(xb.at[idx_v.at[pl.ds(ch * chk, chk)]],
                             rows_v.at[buf], sem)

        def gather_wait(buf, sem):
            pltpu.make_async_copy(xb.at[idx_v.at[pl.ds(0, chk)]],
                                  rows_v.at[buf], sem).wait()

        for b in range(_NBUF):
            gather(b, b, sems[b])

        def body(base, carry):
            for b in range(_NBUF):
                ch = base + b
                gather_wait(b, sems[b])
                for v in range(_CH):
                    for c in range(C2 // _L):
                        sl = pl.ds(c * _L, _L)
                        w = rows_v[b, v * K, sl]
                        se = lax.bitcast_convert_type(
                            lax.shift_left(w, 16), jnp.float32)
                        so = lax.bitcast_convert_type(w, jnp.float32)
                        for k in range(1, K):
                            w = rows_v[b, v * K + k, sl]
                            se = se + lax.bitcast_convert_type(
                                lax.shift_left(w, 16), jnp.float32)
                            so = so + lax.bitcast_convert_type(w, jnp.float32)
                        row = b * _CH + v
                        out_v[row, pl.ds(c * 2 * _L, _L)] = se * (1.0 / K)
                        out_v[row, pl.ds(c * 2 * _L + _L, _L)] = so * (1.0 / K)
                nxt = ch + _NBUF
                @pl.when(nxt < n_ch)
                def _():
                    gather(nxt, b, sems[b])
            pltpu.sync_copy(out_v, out_hbm.at[pl.ds(vbase + base * _CH,
                                                    _NBUF * _CH)])
            return carry

        lax.fori_loop(0, n_ch // _NBUF, lambda i, c: body(i * _NBUF, c), 0)

    return mean_kernel


def _conv_body(x_ref, m_ref, w0t_ref, w1t_ref, b_ref, o_ref):
    acc = lax.dot_general(
        x_ref[0], w0t_ref[...], (((1,), (0,)), ((), ())),
        preferred_element_type=jnp.float32)
    acc = acc + lax.dot_general(
        m_ref[0], w1t_ref[...], (((1,), (0,)), ((), ())),
        preferred_element_type=jnp.float32)
    o_ref[0] = acc + b_ref[...]


def _tc_conv(x_pad, mean_pad, w0t, w1t, b2d, V, BV=2000):
    B, _, C = x_pad.shape
    Cout = w0t.shape[1]
    grid = (B, V // BV)
    return pl.pallas_call(
        _conv_body,
        grid=grid,
        in_specs=[
            pl.BlockSpec((1, BV, C), lambda i, j: (i, j, 0)),
            pl.BlockSpec((1, BV, C), lambda i, j: (i, j, 0)),
            pl.BlockSpec((C, Cout), lambda i, j: (0, 0)),
            pl.BlockSpec((C, Cout), lambda i, j: (0, 0)),
            pl.BlockSpec((1, Cout), lambda i, j: (0, 0)),
        ],
        out_specs=pl.BlockSpec((1, BV, Cout), lambda i, j: (i, j, 0)),
        out_shape=jax.ShapeDtypeStruct((B, V, Cout), jnp.float32),
    )(x_pad, mean_pad, w0t, w1t, b2d)


def kernel(x, mesh, W, b):
    B, C, V = x.shape
    K = mesh.shape[-1]
    Vp = 10240 if V == 10000 else ((V + _NW * 16 - 1) // (_NW * 16)) * (_NW * 16)
    x_t = jnp.transpose(x, (0, 2, 1)).astype(jnp.bfloat16)
    x_pad = jnp.pad(x_t, ((0, 0), (0, Vp - V), (0, 0)))
    mesh_pad = jnp.pad(mesh, ((0, 0), (0, Vp - V), (0, 0)))
    x_words = lax.bitcast_convert_type(
        x_pad.reshape(B * Vp, C // 2, 2), jnp.int32)
    mean_pad = _make_mean_kernel(B, V, Vp, C, K)(
        x_words, mesh_pad.reshape(B * Vp * K))
    w0t = jnp.transpose(W[:, :, 0, 0]).astype(jnp.bfloat16)
    perm = np.arange(C).reshape(C // 32, 16, 2).transpose(0, 2, 1).reshape(C)
    w1t = jnp.transpose(W[:, :, 0, 1])[perm]
    out_t = _tc_conv(x_pad, mean_pad.reshape(B, Vp, C), w0t, w1t,
                     b.reshape(1, -1), V)
    out = jnp.transpose(out_t, (0, 2, 1))
    return out[..., None]
